# Initial kernel scaffold; baseline (speedup 1.0000x reference)
import jax
import jax.numpy as jnp
from jax.experimental import pallas as pl


def kernel(predictions, targets, input_size):
    raise NotImplementedError("write your pallas kernel here")


# speedup vs baseline: 1.8055x; 1.8055x over previous
import functools
import math

import jax
import jax.numpy as jnp
from jax import lax
from jax.experimental import pallas as pl
from jax.experimental.pallas import tpu as pltpu
from jax.experimental.pallas import tpu_sc as plsc

B = 32
N = 8400
C = 80
D = 85
M = 200
MPAD = 256
STRIDES = (8, 16, 32)
GRIDS = (80, 40, 20)
OFFSETS = (0, 6400, 8000)
EPS = 1e-7
NC = 2
NS = 16
NW = NC * NS
SLOTS_PER_W = 32
SLOTS = NW * SLOTS_PER_W
DENSE_BLK = 840
DENSE_STEPS = (B * N) // DENSE_BLK


OBJ_PER_W = (B * N) // NW
OBJ_CH = 66
OBJ_PAD = OBJ_CH * 128
DPAD = 88
ROW_CH = (SLOTS_PER_W * DPAD) // 128
ROW_PAD = ROW_CH * 128
FLAT = B * N * D
MAX_INFLIGHT = 22


@functools.partial(
    pl.kernel,
    out_type=(
        jax.ShapeDtypeStruct((NW, ROW_PAD), jnp.float32),
        jax.ShapeDtypeStruct((B * N,), jnp.float32),
    ),
    mesh=plsc.VectorSubcoreMesh(core_axis_name="c", subcore_axis_name="s"),
    compiler_params=pltpu.CompilerParams(use_tc_tiling_on_sc=False),
    scratch_types=[
        pltpu.VMEM((3 * MPAD,), jnp.float32),
        pltpu.VMEM((SLOTS_PER_W,), jnp.int32),
        pltpu.VMEM((ROW_PAD,), jnp.int32),
        pltpu.VMEM((ROW_PAD,), jnp.float32),
        pltpu.VMEM((OBJ_PAD,), jnp.int32),
        pltpu.VMEM((OBJ_PAD,), jnp.float32),
        pltpu.VMEM((16,), jnp.float32),
        pltpu.SemaphoreType.DMA,
        pltpu.SemaphoreType.DMA,
    ],
)
def _sc_gather(pred_hbm, tcols_hbm, isz_hbm, rows_hbm, obj_hbm, tv, idx_v,
               ridx_v, rows_v, oidx_v, obj_v, isz_v, semr, semo):
    wid = lax.axis_index("s") * NC + lax.axis_index("c")
    obase = wid * OBJ_PER_W
    pltpu.sync_copy(tcols_hbm, tv)
    pltpu.sync_copy(isz_hbm, isz_v)
    isz = isz_v[...]
    lanes = lax.iota(jnp.int32, 16)

    for j2 in range(SLOTS_PER_W // 16):
        base_k = wid * SLOTS_PER_W + j2 * 16
        base_m = base_k & 255
        k = base_k + lanes
        s_id = jnp.minimum(k >> 8, 2)
        tb = tv[pl.ds(base_m, 16)]
        tx = tv[pl.ds(MPAD + base_m, 16)]
        ty = tv[pl.ds(2 * MPAD + base_m, 16)]
        sf = jnp.where(s_id == 0, jnp.float32(8.0),
                       jnp.where(s_id == 1, jnp.float32(16.0),
                                 jnp.float32(32.0)))
        g = jnp.where(s_id == 0, 80, jnp.where(s_id == 1, 40, 20))
        off = jnp.where(s_id == 0, 0, jnp.where(s_id == 1, 6400, 8000))
        gx = jnp.clip(((tx * isz) / sf).astype(jnp.int32), 0, g - 1)
        gy = jnp.clip(((ty * isz) / sf).astype(jnp.int32), 0, g - 1)
        idx = tb.astype(jnp.int32) * N + gy * g + gx + off
        idx_v[pl.ds(j2 * 16, 16)] = idx

    def _row_body(e, _):
        r0 = idx_v[pl.ds(0, 16)]
        r1 = idx_v[pl.ds(16, 16)]
        v0 = jnp.minimum(r0 * D + e, FLAT - 1)
        v1 = jnp.minimum(r1 * D + e, FLAT - 1)
        ridx_v[pl.ds(e * SLOTS_PER_W, 16)] = v0
        ridx_v[pl.ds(e * SLOTS_PER_W + 16, 16)] = v1
        return 0

    lax.fori_loop(0, DPAD, _row_body, 0)

    def _obj_body(gi, _):
        i = jnp.minimum(obase + gi * 16 + lanes, B * N - 1)
        oidx_v[pl.ds(gi * 16, 16)] = i * D + 4
        return 0

    lax.fori_loop(0, OBJ_PAD // 16, _obj_body, 0)

    for c in range(ROW_CH):
        pltpu.async_copy(pred_hbm.at[ridx_v.at[pl.ds(c * 128, 128)]],
                         rows_v.at[pl.ds(c * 128, 128)], semr)
    for c in range(MAX_INFLIGHT):
        pltpu.async_copy(pred_hbm.at[oidx_v.at[pl.ds(c * 128, 128)]],
                         obj_v.at[pl.ds(c * 128, 128)], semo)
    for c in range(MAX_INFLIGHT, OBJ_CH):
        d = c - MAX_INFLIGHT
        pltpu.make_async_copy(pred_hbm.at[oidx_v.at[pl.ds(d * 128, 128)]],
                              obj_v.at[pl.ds(d * 128, 128)], semo).wait()
        pltpu.async_copy(pred_hbm.at[oidx_v.at[pl.ds(c * 128, 128)]],
                         obj_v.at[pl.ds(c * 128, 128)], semo)
    for c in range(OBJ_CH - MAX_INFLIGHT, OBJ_CH):
        pltpu.make_async_copy(pred_hbm.at[oidx_v.at[pl.ds(c * 128, 128)]],
                              obj_v.at[pl.ds(c * 128, 128)], semo).wait()
    for c in range(ROW_CH):
        pltpu.make_async_copy(pred_hbm.at[ridx_v.at[pl.ds(c * 128, 128)]],
                              rows_v.at[pl.ds(c * 128, 128)], semr).wait()

    pltpu.sync_copy(rows_v, rows_hbm.at[wid])
    pltpu.sync_copy(obj_v.at[pl.ds(0, OBJ_PER_W)],
                    obj_hbm.at[pl.ds(obase, OBJ_PER_W)])


def _softplus(v):
    return jnp.maximum(v, 0.0) + jnp.log1p(jnp.exp(-jnp.abs(v)))


def _atan_pos(x):
    t3 = 2.414213562373095
    t1 = 0.4142135623730950
    c = jnp.where(x > t3, -1.0 / x, jnp.where(x > t1, (x - 1.0) / (x + 1.0),
                                              x))
    base = jnp.where(x > t3, math.pi / 2,
                     jnp.where(x > t1, math.pi / 4, 0.0))
    zz = c * c
    p = (((8.05374449538e-2 * zz - 1.38776856032e-1) * zz
          + 1.99777106478e-1) * zz - 3.33329491539e-1) * zz * c + c
    return base + p


def _fin_body(rows_ref, tgt_ref, obj_ref, isz_ref, out_ref):
    isz = isz_ref[0, 0]
    dsum = jnp.sum(_softplus(obj_ref[...]))
    t = tgt_ref[...]
    bcol = t[:, 0].astype(jnp.int32)
    ccol = t[:, 1].astype(jnp.int32)
    tx, ty, tw, th = t[:, 2], t[:, 3], t[:, 4], t[:, 5]

    mi = lax.broadcasted_iota(jnp.int32, (MPAD, MPAD), 0)
    mj = lax.broadcasted_iota(jnp.int32, (MPAD, MPAD), 1)
    later = mj > mi
    validj = (mj < M) & later
    valid_f = (lax.iota(jnp.int32, MPAD) < M).astype(jnp.float32)
    eq_c = ccol[:, None] == ccol[None, :]
    cls_iota = lax.broadcasted_iota(jnp.int32, (C, MPAD), 0)
    onehot = jnp.where(ccol[None, :] == cls_iota, 1.0, 0.0)

    num_pos = jnp.float32(0.0)
    s_obj = jnp.float32(0.0)
    s_box = jnp.float32(0.0)
    s_sp = jnp.float32(0.0)
    s_lg = jnp.float32(0.0)
    for si in range(3):
        s = float(STRIDES[si])
        g = GRIDS[si]
        off = OFFSETS[si]
        gx = jnp.clip(((tx * isz) / jnp.float32(s)).astype(jnp.int32), 0,
                      g - 1)
        gy = jnp.clip(((ty * isz) / jnp.float32(s)).astype(jnp.int32), 0,
                      g - 1)
        idx = bcol * N + gy * g + gx + off
        eq = idx[:, None] == idx[None, :]
        dup = jnp.where(eq & validj, 1.0, 0.0)
        win_f = valid_f * jnp.where(jnp.sum(dup, axis=1) > 0.0, 0.0, 1.0)
        dupc = jnp.where(eq & eq_c & validj, 1.0, 0.0)
        keep_f = valid_f * jnp.where(jnp.sum(dupc, axis=1) > 0.0, 0.0, 1.0)

        sl = slice(si * MPAD, (si + 1) * MPAD)
        lx, ly = rows_ref[0, sl], rows_ref[1, sl]
        lw, lh = rows_ref[2, sl], rows_ref[3, sl]
        obj = rows_ref[4, sl]
        cls = rows_ref[5:5 + C, sl]

        px = (1.0 / (1.0 + jnp.exp(-lx)) + gx.astype(jnp.float32)) * s / isz
        py = (1.0 / (1.0 + jnp.exp(-ly)) + gy.astype(jnp.float32)) * s / isz
        pw = 1.0 / (1.0 + jnp.exp(-lw))
        ph = 1.0 / (1.0 + jnp.exp(-lh))

        px1, px2 = px - pw / 2, px + pw / 2
        py1, py2 = py - ph / 2, py + ph / 2
        tx1, tx2 = tx - tw / 2, tx + tw / 2
        ty1, ty2 = ty - th / 2, ty + th / 2
        ix1 = jnp.maximum(px1, tx1)
        iy1 = jnp.maximum(py1, ty1)
        ix2 = jnp.minimum(px2, tx2)
        iy2 = jnp.minimum(py2, ty2)
        inter = jnp.clip(ix2 - ix1, 0.0) * jnp.clip(iy2 - iy1, 0.0)
        pa = (px2 - px1) * (py2 - py1)
        ta = (tx2 - tx1) * (ty2 - ty1)
        union = pa + ta - inter + EPS
        iou = inter / union
        ex1 = jnp.minimum(px1, tx1)
        ey1 = jnp.minimum(py1, ty1)
        ex2 = jnp.maximum(px2, tx2)
        ey2 = jnp.maximum(py2, ty2)
        c2 = (ex2 - ex1) ** 2 + (ey2 - ey1) ** 2 + EPS
        pcx, pcy = (px1 + px2) / 2, (py1 + py2) / 2
        tcx, tcy = (tx1 + tx2) / 2, (ty1 + ty2) / 2
        rho2 = (pcx - tcx) ** 2 + (pcy - tcy) ** 2
        v = (4.0 / math.pi ** 2) * (_atan_pos(tw / (th + EPS)) -
                                    _atan_pos(pw / (ph + EPS))) ** 2
        alpha = v / (1.0 - iou + v + EPS)
        ciou = iou - rho2 / c2 - alpha * v

        num_pos += jnp.sum(win_f)
        s_obj += jnp.sum(win_f * obj)
        s_box += jnp.sum(win_f * (1.0 - ciou))
        s_sp += jnp.sum(win_f[None, :] * _softplus(cls))
        s_lg += jnp.sum(keep_f[None, :] * onehot * cls)

    obj_loss = (dsum - s_obj) / jnp.float32(B * N)
    box_loss = s_box / jnp.maximum(num_pos, 1.0)
    cls_loss = (s_sp - s_lg) / jnp.maximum(num_pos * C, 1.0)
    out_ref[0, 0] = 5.0 * box_loss + obj_loss + cls_loss


_finalize = pl.pallas_call(
    _fin_body,
    in_specs=[
        pl.BlockSpec((DPAD, SLOTS), lambda: (0, 0)),
        pl.BlockSpec((MPAD, 8), lambda: (0, 0)),
        pl.BlockSpec(((B * N) // 128, 128), lambda: (0, 0)),
        pl.BlockSpec(memory_space=pltpu.SMEM),
    ],
    out_specs=pl.BlockSpec(memory_space=pltpu.SMEM),
    out_shape=jax.ShapeDtypeStruct((1, 1), jnp.float32),
)


def kernel(predictions, targets, input_size):
    iszf = jnp.asarray(input_size, jnp.float32)
    isz16 = jnp.full((16,), iszf, jnp.float32)
    isz11 = jnp.reshape(iszf, (1, 1))
    tcols = (jnp.zeros((3 * MPAD,), jnp.float32)
             .at[0:M].set(targets[:, 0])
             .at[MPAD:MPAD + M].set(targets[:, 2])
             .at[2 * MPAD:2 * MPAD + M].set(targets[:, 3]))
    tpad = jnp.zeros((MPAD, 8), jnp.float32).at[:M, :6].set(targets)
    rows, obj = _sc_gather(predictions.reshape(-1), tcols, isz16)
    rows_t = rows.reshape(NW, DPAD, SLOTS_PER_W).transpose(1, 0, 2)
    rows_t = rows_t.reshape(DPAD, SLOTS)
    total = _finalize(rows_t, tpad, obj.reshape((B * N) // 128, 128), isz11)
    return total.reshape(())
